# baseline (speedup 1.0000x reference)
import functools

import jax
import jax.numpy as jnp
from jax import lax
from jax.experimental import pallas as pl
from jax.experimental.pallas import tpu as pltpu
from jax.experimental.pallas import tpu_sc as plsc

NC = 2
NS = 16
NW = NC * NS
NBUF = 4
LOOK = 2


def _sc_kernel(N, E, K, V, D, C, EPT):
    per_w = N // 2 // NW
    n_chunks = per_w // C
    assert per_w % C == 0 and n_chunks % NBUF == 0 and n_chunks >= 2 * NBUF
    E_pad = ((E + EPT - 1) // EPT) * EPT
    half = EPT // 2
    mesh = plsc.VectorSubcoreMesh(
        core_axis_name="c", subcore_axis_name="s",
        num_cores=NC, num_subcores=NS)

    @functools.partial(
        pl.kernel,
        mesh=mesh,
        out_type=[
            jax.ShapeDtypeStruct((N // 2, 2 * D), jnp.float32),
            jax.ShapeDtypeStruct((E_pad, D), jnp.float32),
        ],
        scratch_types=[
            pltpu.VMEM((half * K,), jnp.int32),
            pltpu.VMEM((half * K, D), jnp.float32),
            pltpu.VMEM((per_w,), jnp.int32),
            pltpu.VMEM((per_w,), jnp.int32),
            pltpu.VMEM((NBUF, C, D), jnp.float32),
            pltpu.VMEM((NBUF, C, D), jnp.float32),
            pltpu.VMEM((EPT, D), jnp.float32),
            pltpu.SemaphoreType.DMA,
            pltpu.SemaphoreType.DMA((NBUF,)),
            pltpu.SemaphoreType.DMA((NBUF,)),
        ],
        compiler_params=pltpu.CompilerParams(use_tc_tiling_on_sc=False),
    )
    def k(ev_hbm, od_hbm, e2s_hbm, table_hbm, out_hbm, emb_hbm, eidx_v,
          s1rows, idx_ev, idx_od, rows_ev, rows_od, emb_v, isem, gsem,
          wsem):
        c = lax.axis_index("c")
        s = lax.axis_index("s")
        wid = s * NC + c
        woff = wid * per_w

        ev_copy = pltpu.async_copy(ev_hbm.at[pl.ds(woff, per_w)], idx_ev,
                                   isem)
        od_copy = pltpu.async_copy(od_hbm.at[pl.ds(woff, per_w)], idx_od,
                                   isem)

        base = jnp.minimum(s * EPT, E - EPT)
        for h in range(2):
            hbase = base + h * half
            pltpu.sync_copy(e2s_hbm.at[pl.ds(hbase * K, half * K)], eidx_v)
            pltpu.async_copy(table_hbm.at[eidx_v], s1rows, gsem.at[0]).wait()

            def env_body(e, _):
                for d in range(D // 16):
                    sl = pl.ds(d * 16, 16)
                    acc = s1rows[e * K, sl]
                    for j in range(1, K):
                        acc = acc + s1rows[e * K + j, sl]
                    emb_v[h * half + e, sl] = acc
                return 0

            lax.fori_loop(0, half, env_body, 0)
        pltpu.sync_copy(emb_v, emb_hbm.at[pl.ds(base, EPT)])
        plsc.subcore_barrier()
        ev_copy.wait()
        od_copy.wait()

        def start_gather(i, b):
            d1 = pltpu.async_copy(
                emb_hbm.at[idx_ev.at[pl.ds(i * C, C)]],
                rows_ev.at[b], gsem.at[b])
            d2 = pltpu.async_copy(
                emb_hbm.at[idx_od.at[pl.ds(i * C, C)]],
                rows_od.at[b], gsem.at[b])
            return d1, d2

        def wait_gather(i, b):
            pltpu.make_async_copy(
                emb_hbm.at[idx_ev.at[pl.ds(i * C, C)]],
                rows_ev.at[b], gsem.at[b]).wait()
            pltpu.make_async_copy(
                emb_hbm.at[idx_od.at[pl.ds(i * C, C)]],
                rows_od.at[b], gsem.at[b]).wait()

        def _wslices(i):
            blk = out_hbm.at[pl.ds(woff + i * C, C)]
            return blk.at[:, pl.ds(0, D)], blk.at[:, pl.ds(D, D)]

        def start_write(i, b):
            dst_ev, dst_od = _wslices(i)
            pltpu.async_copy(rows_ev.at[b], dst_ev, wsem.at[b])
            pltpu.async_copy(rows_od.at[b], dst_od, wsem.at[b])

        def wait_write(i, b):
            dst_ev, dst_od = _wslices(i)
            pltpu.make_async_copy(rows_ev.at[b], dst_ev, wsem.at[b]).wait()
            pltpu.make_async_copy(rows_od.at[b], dst_od, wsem.at[b]).wait()

        for b in range(LOOK):
            start_gather(b, b)
        for b in range(NBUF):
            i = b
            wait_gather(i, b)
            start_write(i, b)
            if b + LOOK < NBUF:
                start_gather(i + LOOK, b + LOOK)
            else:
                wait_write(i - LOOK, (b + LOOK) % NBUF)
                start_gather(i + LOOK, (b + LOOK) % NBUF)

        def group(g, _):
            for b in range(NBUF):
                i = g * NBUF + b
                wait_gather(i, b)
                start_write(i, b)
                j = i + LOOK
                bj = (b + LOOK) % NBUF
                wait_write(j - NBUF, bj)

                @pl.when(j < n_chunks)
                def _():
                    start_gather(j, bj)

            return 0

        lax.fori_loop(1, n_chunks // NBUF, group, 0)

        for t in range(LOOK):
            i = n_chunks - LOOK + t
            wait_write(i, i % NBUF)

    return k


def _tc_relayout(B, L, D, BR):
    L2 = L // 2
    n_blk = BR * L2

    def body(in_ref, out_ref):
        x = in_ref[...]
        out_ref[:, :L2, :] = x[:, :D].reshape(BR, L2, D)
        out_ref[:, L2:, :] = x[:, D:].reshape(BR, L2, D)

    return pl.pallas_call(
        body,
        grid=(B // BR,),
        in_specs=[pl.BlockSpec((n_blk, 2 * D), lambda i: (i, 0))],
        out_specs=pl.BlockSpec((BR, L, D), lambda i: (i, 0, 0)),
        out_shape=jax.ShapeDtypeStruct((B, L, D), jnp.float32),
    )


def kernel(env_ids, env2sketchs, sketch_table):
    B, L = env_ids.shape
    E, K = env2sketchs.shape
    V, D = sketch_table.shape
    N = B * L
    ids32 = env_ids.astype(jnp.int32)
    ids_ev = ids32[:, :L // 2].reshape(-1)
    ids_od = ids32[:, L // 2:].reshape(-1)
    e2s = env2sketchs.reshape(-1).astype(jnp.int32)
    table = sketch_table.astype(jnp.float32)
    sc = _sc_kernel(N, E, K, V, D, C=160, EPT=64)
    pairs, _ = sc(ids_ev, ids_od, e2s, table)
    return _tc_relayout(B, L, D, BR=32)(pairs)
